# Initial kernel scaffold; baseline (speedup 1.0000x reference)
import jax
import jax.numpy as jnp
from jax.experimental import pallas as pl


def kernel(inSen, adj, emb, W, a, Wc, bc):
    raise NotImplementedError("write your pallas kernel here")


# speedup vs baseline: 1.4153x; 1.4153x over previous
import functools

import jax
import jax.numpy as jnp
from jax import lax
from jax.experimental import pallas as pl
from jax.experimental.pallas import tpu as pltpu
from jax.experimental.pallas import tpu_sc as plsc

_N = 4096
_EDIM = 64
_WFEAT = 64
_LABELS = 2
_SLOPE = 0.01
_BR = 256
_NSTRIPS = _N // _BR

_NC = 2
_NS = 16
_NW = _NC * _NS
_BPW = _N // _NW


def _sc_gather_body(table_hbm, idx_hbm, out_hbm, idx_v, rows_v, sem):
    wid = lax.axis_index("s") * _NC + lax.axis_index("c")
    base = wid * _BPW
    pltpu.sync_copy(idx_hbm.at[pl.ds(base, _BPW)], idx_v)
    pltpu.async_copy(table_hbm.at[idx_v], rows_v, sem).wait()
    pltpu.sync_copy(rows_v, out_hbm.at[pl.ds(base, _BPW)])


def _sc_gather(table, idx):
    mesh = plsc.VectorSubcoreMesh(core_axis_name="c", subcore_axis_name="s")
    fn = functools.partial(
        pl.kernel,
        mesh=mesh,
        out_type=jax.ShapeDtypeStruct((_N, _EDIM), jnp.float32),
        scratch_types=[
            pltpu.VMEM((_BPW,), jnp.int32),
            pltpu.VMEM((_BPW, _EDIM), jnp.float32),
            pltpu.SemaphoreType.DMA,
        ],
        compiler_params=pltpu.CompilerParams(use_tc_tiling_on_sc=False),
    )(_sc_gather_body)
    return fn(table, idx)


def _tc_body(adj_ref, words_ref, w_ref, a1_ref, a2r_ref, wc_ref, bc_ref,
             att_ref, sent_ref, pool_ref, label_ref, wh_s, f2r_s, pool_s):
    i = pl.program_id(0)

    @pl.when(i == 0)
    def _init():
        wh = jnp.dot(words_ref[...], w_ref[...],
                     preferred_element_type=jnp.float32)
        wh_s[...] = wh
        f2r_s[...] = lax.dot_general(
            a2r_ref[...], wh, (((1,), (1,)), ((), ())),
            preferred_element_type=jnp.float32)
        pool_s[...] = jnp.zeros_like(pool_s)

    row0 = pl.multiple_of(i * _BR, _BR)
    wh_strip = wh_s[pl.ds(row0, _BR), :]
    f1 = jnp.dot(wh_strip, a1_ref[...],
                 preferred_element_type=jnp.float32)
    e = f1 + f2r_s[...]
    e = jnp.where(e >= 0, e, _SLOPE * e)
    e = jnp.where(adj_ref[...] > 0, e, jnp.float32(-9e15))
    m = jnp.max(e, axis=1, keepdims=True)
    p = jnp.exp(e - m)
    s = jnp.sum(p, axis=1, keepdims=True)
    att = p / s
    att_ref[...] = att
    h = jnp.dot(att, wh_s[...], preferred_element_type=jnp.float32)
    sent = jnp.where(h > 0, h, jnp.exp(h) - 1.0)
    sent_ref[...] = sent
    pool_s[...] += jnp.sum(sent, axis=0, keepdims=True)

    @pl.when(i == _NSTRIPS - 1)
    def _fin():
        pool = pool_s[...] / jnp.float32(_N)
        pool_ref[...] = pool
        logits = jnp.dot(pool, wc_ref[...],
                         preferred_element_type=jnp.float32) + bc_ref[...]
        lm = jnp.max(logits, axis=1, keepdims=True)
        ex = jnp.exp(logits - lm)
        label_ref[...] = ex / jnp.sum(ex, axis=1, keepdims=True)


def _tc_main(adj, words, W, a1, a2r, Wc, bcr):
    return pl.pallas_call(
        _tc_body,
        grid=(_NSTRIPS,),
        in_specs=[
            pl.BlockSpec((_BR, _N), lambda i: (i, 0)),
            pl.BlockSpec((_N, _EDIM), lambda i: (0, 0)),
            pl.BlockSpec((_EDIM, _WFEAT), lambda i: (0, 0)),
            pl.BlockSpec((_WFEAT, 1), lambda i: (0, 0)),
            pl.BlockSpec((1, _WFEAT), lambda i: (0, 0)),
            pl.BlockSpec((_WFEAT, _LABELS), lambda i: (0, 0)),
            pl.BlockSpec((1, _LABELS), lambda i: (0, 0)),
        ],
        out_specs=[
            pl.BlockSpec((_BR, _N), lambda i: (i, 0)),
            pl.BlockSpec((_BR, _WFEAT), lambda i: (i, 0)),
            pl.BlockSpec((1, _WFEAT), lambda i: (0, 0)),
            pl.BlockSpec((1, _LABELS), lambda i: (0, 0)),
        ],
        out_shape=[
            jax.ShapeDtypeStruct((_N, _N), jnp.float32),
            jax.ShapeDtypeStruct((_N, _WFEAT), jnp.float32),
            jax.ShapeDtypeStruct((1, _WFEAT), jnp.float32),
            jax.ShapeDtypeStruct((1, _LABELS), jnp.float32),
        ],
        scratch_shapes=[
            pltpu.VMEM((_N, _WFEAT), jnp.float32),
            pltpu.VMEM((1, _N), jnp.float32),
            pltpu.VMEM((1, _WFEAT), jnp.float32),
        ],
        compiler_params=pltpu.CompilerParams(
            dimension_semantics=("arbitrary",)),
    )(adj, words, W, a1, a2r, Wc, bcr)


def kernel(inSen, adj, emb, W, a, Wc, bc):
    words = _sc_gather(emb, inSen.astype(jnp.int32))
    a1 = a[:_WFEAT]
    a2r = a[_WFEAT:].reshape(1, _WFEAT)
    bcr = bc.reshape(1, _LABELS)
    att, sent, pool, label = _tc_main(adj, words, W, a1, a2r, Wc, bcr)
    return (pool.reshape(_WFEAT), att, sent, label.reshape(_LABELS))
